# Initial kernel scaffold; baseline (speedup 1.0000x reference)
import functools

import jax
import jax.numpy as jnp
from jax import lax
from jax.experimental import pallas as pl
from jax.experimental.pallas import tpu as pltpu
from jax.experimental.pallas import tpu_sc as plsc

_K = 16
_L = 16
_B = 4
_N = 4096
_NW = 32
_ROWS_PER_W = (_B * _N) // _NW
_TILES_PER_B = _N // _ROWS_PER_W
_NV = _N // _L


def _knn_body(px_hbm, py_hbm, pz_hbm, out_hbm, px_v, py_v, pz_v, sq_v, out_v):
    cid = lax.axis_index("c")
    sid = lax.axis_index("s")
    wid = sid * 2 + cid
    b = wid // _TILES_PER_B
    r0 = (wid % _TILES_PER_B) * _ROWS_PER_W

    pltpu.sync_copy(px_hbm.at[b], px_v)
    pltpu.sync_copy(py_hbm.at[b], py_v)
    pltpu.sync_copy(pz_hbm.at[b], pz_v)

    def sq_body(j, carry):
        x = px_v[pl.ds(j * _L, _L)]
        y = py_v[pl.ds(j * _L, _L)]
        z = pz_v[pl.ds(j * _L, _L)]
        sq_v[pl.ds(j * _L, _L)] = x * x + y * y + z * z
        return carry

    lax.fori_loop(0, _NV, sq_body, 0)

    inf = jnp.float32(jnp.inf)

    def row_body(r, carry):
        i = r0 + r
        xi = px_v[i]
        yi = py_v[i]
        zi = pz_v[i]
        sqi = sq_v[i]

        def scan_body(j, st):
            t_val, t_idx, thr = st
            x = px_v[pl.ds(j * _L, _L)]
            y = py_v[pl.ds(j * _L, _L)]
            z = pz_v[pl.ds(j * _L, _L)]
            s = sq_v[pl.ds(j * _L, _L)]
            dot = x * xi + y * yi + z * zi
            d = (sqi + s) - 2.0 * dot
            m = d < thr
            cnt = plsc.all_reduce_population_count(m)

            def do_merge(_):
                cidx = lax.iota(jnp.int32, _L) + j * _L
                csv, csi = plsc.sort_key_val(d, cidx)
                rcv = lax.rev(csv, (0,))
                rci = lax.rev(csi, (0,))
                keep = t_val <= rcv
                lov = jnp.where(keep, t_val, rcv)
                loi = jnp.where(keep, t_idx, rci)
                nv, ni = plsc.sort_key_val(lov, loi)
                return nv, ni, jnp.max(nv)

            def no_merge(_):
                return t_val, t_idx, thr

            return lax.cond(cnt[0] > 0, do_merge, no_merge, 0)

        t_val0 = jnp.full((_L,), inf, jnp.float32)
        t_idx0 = jnp.zeros((_L,), jnp.int32)
        tv, ti, _ = lax.fori_loop(0, _NV, scan_body, (t_val0, t_idx0, inf))
        out_v[r] = ti
        return carry

    lax.fori_loop(0, _ROWS_PER_W, row_body, 0)
    pltpu.sync_copy(out_v, out_hbm.at[pl.ds(wid * _ROWS_PER_W, _ROWS_PER_W)])


_knn = functools.partial(
    pl.kernel,
    out_type=jax.ShapeDtypeStruct((_B * _N, _K), jnp.int32),
    mesh=plsc.VectorSubcoreMesh(core_axis_name="c", subcore_axis_name="s"),
    scratch_types=[
        pltpu.VMEM((_N,), jnp.float32),
        pltpu.VMEM((_N,), jnp.float32),
        pltpu.VMEM((_N,), jnp.float32),
        pltpu.VMEM((_N,), jnp.float32),
        pltpu.VMEM((_ROWS_PER_W, _K), jnp.int32),
    ],
)(_knn_body)


@jax.jit
def kernel(points):
    px = points[..., 0]
    py = points[..., 1]
    pz = points[..., 2]
    idx = _knn(px, py, pz)
    return idx.reshape(_B, _N, _K)


# speedup vs baseline: 3.1821x; 3.1821x over previous
import functools

import jax
import jax.numpy as jnp
from jax import lax
from jax.experimental import pallas as pl
from jax.experimental.pallas import tpu as pltpu
from jax.experimental.pallas import tpu_sc as plsc

_K = 16
_L = 16
_B = 4
_N = 4096
_NW = 32
_ROWS_PER_W = (_B * _N) // _NW
_TILES_PER_B = _N // _ROWS_PER_W
_NV = _N // _L


def _knn_body(px_hbm, py_hbm, pz_hbm, out_hbm, px_v, py_v, pz_v, sq_v, out_v):
    cid = lax.axis_index("c")
    sid = lax.axis_index("s")
    wid = sid * 2 + cid
    b = wid // _TILES_PER_B
    r0 = (wid % _TILES_PER_B) * _ROWS_PER_W

    pltpu.sync_copy(px_hbm.at[b], px_v)
    pltpu.sync_copy(py_hbm.at[b], py_v)
    pltpu.sync_copy(pz_hbm.at[b], pz_v)

    def _bf16_round(v):
        u = plsc.bitcast(v, jnp.int32)
        bias = jnp.int32(0x7FFF) + ((u >> 16) & jnp.int32(1))
        return plsc.bitcast((u + bias) & jnp.int32(-65536), jnp.float32)

    def sq_body(j, carry):
        x = px_v[pl.ds(j * _L, _L)]
        y = py_v[pl.ds(j * _L, _L)]
        z = pz_v[pl.ds(j * _L, _L)]
        sq_v[pl.ds(j * _L, _L)] = (x * x + y * y) + z * z
        px_v[pl.ds(j * _L, _L)] = _bf16_round(x)
        py_v[pl.ds(j * _L, _L)] = _bf16_round(y)
        pz_v[pl.ds(j * _L, _L)] = _bf16_round(z)
        return carry

    lax.fori_loop(0, _NV, sq_body, 0)

    inf = jnp.float32(jnp.inf)

    def group_body(g, carry):
        base = r0 + g * _L
        qx = px_v[pl.ds(base, _L)]
        qy = py_v[pl.ds(base, _L)]
        qz = pz_v[pl.ds(base, _L)]
        qs = sq_v[pl.ds(base, _L)]
        for u in range(_L):
            xi = qx[u]
            yi = qy[u]
            zi = qz[u]
            sqi = qs[u]

            def scan_body(j, st, xi=xi, yi=yi, zi=zi, sqi=sqi):
                t_val, t_idx, thr = st
                x = px_v[pl.ds(j * _L, _L)]
                y = py_v[pl.ds(j * _L, _L)]
                z = pz_v[pl.ds(j * _L, _L)]
                s = sq_v[pl.ds(j * _L, _L)]
                dot = x * xi + y * yi + z * zi
                d = (sqi + s) - 2.0 * dot
                dmin = jnp.min(d)

                def do_merge(_):
                    cidx = lax.iota(jnp.int32, _L) + j * _L
                    csv, csi = plsc.sort_key_val(d, cidx)
                    rcv = lax.rev(csv, (0,))
                    rci = lax.rev(csi, (0,))
                    keep = t_val <= rcv
                    lov = jnp.where(keep, t_val, rcv)
                    loi = jnp.where(keep, t_idx, rci)
                    nv, ni = plsc.sort_key_val(lov, loi)
                    return nv, ni, jnp.max(nv)

                def no_merge(_):
                    return t_val, t_idx, thr

                return lax.cond(dmin < thr, do_merge, no_merge, 0)

            t_val0 = jnp.full((_L,), inf, jnp.float32)
            t_idx0 = jnp.zeros((_L,), jnp.int32)
            tv, ti, _ = lax.fori_loop(0, _NV, scan_body, (t_val0, t_idx0, inf))
            out_v[pl.ds((g * _L + u) * _K, _K)] = ti
        return carry

    lax.fori_loop(0, _ROWS_PER_W // _L, group_body, 0)
    pltpu.sync_copy(
        out_v, out_hbm.at[pl.ds(wid * _ROWS_PER_W * _K, _ROWS_PER_W * _K)]
    )


_knn = functools.partial(
    pl.kernel,
    out_type=jax.ShapeDtypeStruct((_B * _N * _K,), jnp.int32),
    mesh=plsc.VectorSubcoreMesh(core_axis_name="c", subcore_axis_name="s"),
    scratch_types=[
        pltpu.VMEM((_N,), jnp.float32),
        pltpu.VMEM((_N,), jnp.float32),
        pltpu.VMEM((_N,), jnp.float32),
        pltpu.VMEM((_N,), jnp.float32),
        pltpu.VMEM((_ROWS_PER_W * _K,), jnp.int32),
    ],
    compiler_params=pltpu.CompilerParams(needs_layout_passes=False),
)(_knn_body)


@jax.jit
def kernel(points):
    px = points[..., 0]
    py = points[..., 1]
    pz = points[..., 2]
    idx = _knn(px, py, pz)
    return idx.reshape(_B, _N, _K)


# speedup vs baseline: 5.6664x; 1.7807x over previous
import functools

import jax
import jax.numpy as jnp
from jax import lax
from jax.experimental import pallas as pl
from jax.experimental.pallas import tpu as pltpu
from jax.experimental.pallas import tpu_sc as plsc

_K = 16
_L = 16
_B = 4
_N = 4096
_NW = 32
_ROWS_PER_W = (_B * _N) // _NW
_TILES_PER_B = _N // _ROWS_PER_W
_NV = _N // _L
_LAG = 4


def _knn_body(px_hbm, py_hbm, pz_hbm, out_hbm,
              px_v, py_v, pz_v, sq_v, bufd_v, bufi_v, out_v):
    cid = lax.axis_index("c")
    sid = lax.axis_index("s")
    wid = sid * 2 + cid
    b = wid // _TILES_PER_B
    r0 = (wid % _TILES_PER_B) * _ROWS_PER_W

    pltpu.sync_copy(px_hbm.at[b], px_v)
    pltpu.sync_copy(py_hbm.at[b], py_v)
    pltpu.sync_copy(pz_hbm.at[b], pz_v)

    def _bf16_round(v):
        u = plsc.bitcast(v, jnp.int32)
        bias = jnp.int32(0x7FFF) + ((u >> 16) & jnp.int32(1))
        return plsc.bitcast((u + bias) & jnp.int32(-65536), jnp.float32)

    def sq_body(j, carry):
        x = px_v[pl.ds(j * _L, _L)]
        y = py_v[pl.ds(j * _L, _L)]
        z = pz_v[pl.ds(j * _L, _L)]
        sq_v[pl.ds(j * _L, _L)] = (x * x + y * y) + z * z
        px_v[pl.ds(j * _L, _L)] = _bf16_round(x)
        py_v[pl.ds(j * _L, _L)] = _bf16_round(y)
        pz_v[pl.ds(j * _L, _L)] = _bf16_round(z)
        return carry

    lax.fori_loop(0, _NV, sq_body, 0)

    inf = jnp.float32(jnp.inf)
    inf_v = jnp.full((_L,), inf, jnp.float32)
    zero_i = jnp.zeros((_L,), jnp.int32)
    lane15 = jnp.full((_L,), 15, jnp.int32)
    iota = lax.iota(jnp.int32, _L)

    def group_body(g, carry):
        base = r0 + g * _L
        qx = px_v[pl.ds(base, _L)]
        qy = py_v[pl.ds(base, _L)]
        qz = pz_v[pl.ds(base, _L)]
        qs = sq_v[pl.ds(base, _L)]
        for u in range(_L):
            xi = qx[u]
            yi = qy[u]
            zi = qz[u]
            sqi = qs[u]

            def scan_body(j, st, xi=xi, yi=yi, zi=zi, sqi=sqi):
                run_min, cntv, _ = st
                thr = jnp.take_along_axis(plsc.cummax(run_min), lane15, axis=0)
                for k in range(_LAG):
                    jj = j * _LAG + k
                    x = px_v[pl.ds(jj * _L, _L)]
                    y = py_v[pl.ds(jj * _L, _L)]
                    z = pz_v[pl.ds(jj * _L, _L)]
                    s = sq_v[pl.ds(jj * _L, _L)]
                    dot = x * xi + y * yi + z * zi
                    d = (sqi + s) - 2.0 * dot
                    mask = d <= thr
                    cs = plsc.cumsum(mask.astype(jnp.int32))
                    pos = (cntv - 1) + cs
                    plsc.store_scatter(bufd_v, [pos], d, mask=mask)
                    cidx = iota + jj * _L
                    plsc.store_scatter(bufi_v, [pos], cidx, mask=mask)
                    cntv = cntv + jnp.take_along_axis(cs, lane15, axis=0)
                    run_min = jnp.minimum(run_min, d)
                return (run_min, cntv, thr)

            st0 = (inf_v, zero_i, inf_v)
            fin = lax.fori_loop(0, _NV // _LAG, scan_body, st0)
            cnt = fin[1][0]
            bufd_v[pl.ds(cnt, _L)] = inf_v

            def merge_body(m, st):
                t_val, t_idx = st
                dblk = bufd_v[pl.ds(m * _L, _L)]
                iblk = bufi_v[pl.ds(m * _L, _L)]
                csv, csi = plsc.sort_key_val(dblk, iblk)
                rcv = lax.rev(csv, (0,))
                rci = lax.rev(csi, (0,))
                keep = t_val <= rcv
                lov = jnp.where(keep, t_val, rcv)
                loi = jnp.where(keep, t_idx, rci)
                nv, ni = plsc.sort_key_val(lov, loi)
                return (nv, ni)

            mi = (cnt + (_L - 1)) >> 4
            tv, ti = lax.fori_loop(0, mi, merge_body, (inf_v, zero_i))
            out_v[pl.ds((g * _L + u) * _K, _K)] = ti
        return carry

    lax.fori_loop(0, _ROWS_PER_W // _L, group_body, 0)
    pltpu.sync_copy(
        out_v, out_hbm.at[pl.ds(wid * _ROWS_PER_W * _K, _ROWS_PER_W * _K)]
    )


_knn = functools.partial(
    pl.kernel,
    out_type=jax.ShapeDtypeStruct((_B * _N * _K,), jnp.int32),
    mesh=plsc.VectorSubcoreMesh(core_axis_name="c", subcore_axis_name="s"),
    scratch_types=[
        pltpu.VMEM((_N,), jnp.float32),
        pltpu.VMEM((_N,), jnp.float32),
        pltpu.VMEM((_N,), jnp.float32),
        pltpu.VMEM((_N,), jnp.float32),
        pltpu.VMEM((_N + _L,), jnp.float32),
        pltpu.VMEM((_N + _L,), jnp.int32),
        pltpu.VMEM((_ROWS_PER_W * _K,), jnp.int32),
    ],
    compiler_params=pltpu.CompilerParams(needs_layout_passes=False),
)(_knn_body)


@jax.jit
def kernel(points):
    px = points[..., 0]
    py = points[..., 1]
    pz = points[..., 2]
    idx = _knn(px, py, pz)
    return idx.reshape(_B, _N, _K)
